# baseline (speedup 1.0000x reference)
import functools

import jax
import jax.numpy as jnp
from jax import lax
from jax.experimental import pallas as pl
from jax.experimental.pallas import tpu as pltpu
from jax.experimental.pallas import tpu_sc as plsc

N = 10000
E = 320000
T = 16
ENC = 12
H = 128
NUM_LAYERS = 2

NW = 32
EPW = E // NW
CHUNK = 128
C = -(-EPW // (2 * CHUNK)) * 2
PACK = 16384
CPAD = C * CHUNK - EPW
ACC = 10112
ZR = ACC // 16
TRASH = N
BN = 1000
NB = N // BN


@functools.lru_cache(maxsize=None)
def _make_prop(W):
    mesh = plsc.VectorSubcoreMesh(core_axis_name="c", subcore_axis_name="s")

    @functools.partial(
        pl.kernel,
        mesh=mesh,
        out_type=jax.ShapeDtypeStruct((2, ACC, W), jnp.float32),
        scratch_types=[
            pltpu.VMEM((C * CHUNK,), jnp.int32),
            pltpu.VMEM((2 * CHUNK,), jnp.int32),
            pltpu.VMEM((2, CHUNK), jnp.int32),
            pltpu.VMEM((2, CHUNK, W), jnp.float32),
            pltpu.VMEM_SHARED((ACC, W), jnp.float32),
            pltpu.SemaphoreType.DMA,
            pltpu.SemaphoreType.DMA,
        ],
    )
    def prop(xs_hbm, pidx_hbm, zeros_hbm, out_hbm,
             pv, gb, sb, rows, acc, sem0, sem1):
        cid = lax.axis_index("c")
        sid = lax.axis_index("s")
        wid = sid * 2 + cid
        pltpu.sync_copy(pidx_hbm.at[wid], pv)
        pltpu.sync_copy(zeros_hbm, acc.at[pl.ds(sid * ZR, ZR)])
        plsc.subcore_barrier()

        def unpack(j, slot):
            for k in range(CHUNK // 16):
                v = pv[pl.ds(j * CHUNK + 16 * k, 16)]
                gb[pl.ds(slot * CHUNK + 16 * k, 16)] = (
                    lax.shift_right_logical(v, 14))
                sb[slot, pl.ds(16 * k, 16)] = lax.bitwise_and(v, PACK - 1)

        def fire(j, slot, sem):
            unpack(j, slot)
            pltpu.async_copy(
                xs_hbm.at[gb.at[pl.ds(slot * CHUNK, CHUNK)]],
                rows.at[slot], sem)

        fire(0, 0, sem0)
        fire(1, 1, sem1)

        def body(i, carry):
            j0 = 2 * i
            for slot, sem in ((0, sem0), (1, sem1)):
                j = j0 + slot
                pltpu.make_async_copy(
                    xs_hbm.at[pl.ds(0, CHUNK)], rows.at[slot], sem).wait()
                pltpu.sync_copy(rows.at[slot], acc.at[sb.at[slot]], add=True)

                @pl.when(j + 2 < C)
                def _():
                    fire(j + 2, slot, sem)
            return carry

        lax.fori_loop(0, C // 2, body, 0)
        plsc.subcore_barrier()
        pltpu.sync_copy(acc.at[pl.ds(sid * ZR, ZR)],
                        out_hbm.at[cid, pl.ds(sid * ZR, ZR)])

    return prop


def _prep_body(kemb_ref, hdw_ref, hdb_ref, kcv_ref, kcb_ref, ocv_ref, ocb_ref,
               tgv_ref, tgb_ref, fdw_ref, fdb_ref, se0_ref, se1_ref, stw_ref,
               stb_ref, t0h_ref, t0f_ref, uch_ref, ucf_ref, s0_ref, s1_ref):
    hdw = hdw_ref[...]
    fdw = fdw_ref[...]
    dot = lambda a, b: jnp.dot(a, b, preferred_element_type=jnp.float32)
    t0h_ref[...] = dot(kemb_ref[...], hdw[0:128])
    t0f_ref[...] = dot(kemb_ref[...], fdw[0:128])
    kcv = kcv_ref[...]
    kcb = kcb_ref[...]
    ocv = ocv_ref[...]
    ocb = ocb_ref[...]
    u1h = dot(kcv[0:1], hdw[128:256])
    u2h = dot(kcv[1:2], hdw[256:384])
    u3h = dot(ocv[0:1], hdw[384:512])
    u4h = dot(ocv[1:2], hdw[512:640])
    u5h = dot(tgv_ref[...], hdw[640:768])
    ch = (dot(kcb[0:1], hdw[128:256]) + dot(kcb[1:2], hdw[256:384])
          + dot(ocb[0:1], hdw[384:512]) + dot(ocb[1:2], hdw[512:640])
          + dot(tgb_ref[...], hdw[640:768]) + hdb_ref[...])
    zrow = jnp.zeros((2, 128), jnp.float32)
    uch_ref[...] = jnp.concatenate([u1h, u2h, u3h, u4h, u5h, ch, zrow], axis=0)
    u1f = dot(kcv[0:1], fdw[128:256])
    u2f = dot(kcv[1:2], fdw[256:384])
    cf = (dot(kcb[0:1], fdw[128:256]) + dot(kcb[1:2], fdw[256:384])
          + fdb_ref[...])
    zrow5 = jnp.zeros((5, 128), jnp.float32)
    ucf_ref[...] = jnp.concatenate([u1f, u2f, cf, zrow5], axis=0)
    stw = stw_ref[...]
    s0_ref[...] = dot(se0_ref[...], stw[0:128])
    s1_ref[...] = dot(se1_ref[...], stw[128:256]) + stb_ref[...]


def _prep(kemb, hdw, hdb, kcv, kcb, ocv, ocb, tgv, tgb, fdw, fdb,
          se0, se1, stw, stb):
    f32 = jnp.float32
    return pl.pallas_call(
        _prep_body,
        out_shape=(
            jax.ShapeDtypeStruct((56, 128), f32),
            jax.ShapeDtypeStruct((56, 128), f32),
            jax.ShapeDtypeStruct((8, 128), f32),
            jax.ShapeDtypeStruct((8, 128), f32),
            jax.ShapeDtypeStruct((104, 256), f32),
            jax.ShapeDtypeStruct((104, 256), f32),
        ),
    )(kemb, hdw, hdb, kcv, kcb, ocv, ocb, tgv, tgb, fdw, fdb, se0, se1,
      stw, stb)


def _feat_body(kcat_ref, kc_ref, oc_ref, tg_ref, s0_ref, dgo_ref, dgi_ref,
               t0h_ref, t0f_ref, uch_ref, ucf_ref, s0t_ref, s1t_ref,
               hx_ref, fx_ref, h0_ref, h0s_ref, no_ref, ni_ref):
    dot = lambda a, b: jnp.dot(a, b, preferred_element_type=jnp.float32)
    no = lax.rsqrt(jnp.maximum(dgo_ref[0, :, 0] + dgo_ref[1, :, 0], 1.0))
    ni = lax.rsqrt(jnp.maximum(dgi_ref[0, :, 0] + dgi_ref[1, :, 0], 1.0))
    no = no[:, None]
    ni = ni[:, None]
    no_ref[...] = jnp.broadcast_to(no, (BN, 8))
    ni_ref[...] = jnp.broadcast_to(ni, (BN, 8))

    ids = kcat_ref[...]
    kc = kc_ref[...]
    oc = oc_ref[...]
    tg = tg_ref[...]
    uch = uch_ref[...]
    ucf = ucf_ref[...]
    iota56 = lax.broadcasted_iota(jnp.int32, (1, 56), 1)
    for t in range(T):
        oh = (ids[:, t][:, None] == iota56).astype(jnp.float32)
        if t < ENC:
            v = (dot(oh, t0h_ref[...])
                 + kc[:, 2 * t][:, None] * uch[0:1]
                 + kc[:, 2 * t + 1][:, None] * uch[1:2]
                 + oc[:, 2 * t][:, None] * uch[2:3]
                 + oc[:, 2 * t + 1][:, None] * uch[3:4]
                 + tg[:, t][:, None] * uch[4:5]
                 + uch[5:6])
            hx_ref[t] = v * no
        else:
            v = (dot(oh, t0f_ref[...])
                 + kc[:, 2 * t][:, None] * ucf[0:1]
                 + kc[:, 2 * t + 1][:, None] * ucf[1:2]
                 + ucf[2:3])
            fx_ref[t - ENC] = v * no

    s0 = s0_ref[...]
    iota104 = lax.broadcasted_iota(jnp.int32, (1, 104), 1)
    oh0 = (s0[:, 0][:, None] == iota104).astype(jnp.float32)
    oh1 = (s0[:, 1][:, None] == iota104).astype(jnp.float32)
    iv = dot(oh0, s0t_ref[...]) + dot(oh1, s1t_ref[...])
    h00 = iv[:, 0:128]
    h01 = iv[:, 128:256]
    h0_ref[0] = h00
    h0_ref[1] = h01
    h0s_ref[0] = h00 * no
    h0s_ref[1] = h01 * no


def _features(kcat, kc, oc, tg, s0, dgo, dgi, t0h, t0f, uch, ucf, s0t, s1t):
    f32 = jnp.float32
    bs = pl.BlockSpec
    return pl.pallas_call(
        _feat_body,
        grid=(NB,),
        in_specs=[
            bs((BN, T), lambda i: (i, 0)),
            bs((BN, 2 * T), lambda i: (i, 0)),
            bs((BN, 2 * T), lambda i: (i, 0)),
            bs((BN, T), lambda i: (i, 0)),
            bs((BN, 2), lambda i: (i, 0)),
            bs((2, BN, H), lambda i: (0, i, 0)),
            bs((2, BN, H), lambda i: (0, i, 0)),
            bs((56, 128), lambda i: (0, 0)),
            bs((56, 128), lambda i: (0, 0)),
            bs((8, 128), lambda i: (0, 0)),
            bs((8, 128), lambda i: (0, 0)),
            bs((104, 256), lambda i: (0, 0)),
            bs((104, 256), lambda i: (0, 0)),
        ],
        out_specs=[
            bs((ENC, BN, H), lambda i: (0, i, 0)),
            bs((T - ENC, BN, H), lambda i: (0, i, 0)),
            bs((2, BN, H), lambda i: (0, i, 0)),
            bs((2, BN, H), lambda i: (0, i, 0)),
            bs((BN, 8), lambda i: (i, 0)),
            bs((BN, 8), lambda i: (i, 0)),
        ],
        out_shape=(
            jax.ShapeDtypeStruct((ENC, N, H), f32),
            jax.ShapeDtypeStruct((T - ENC, N, H), f32),
            jax.ShapeDtypeStruct((2, N, H), f32),
            jax.ShapeDtypeStruct((2, N, H), f32),
            jax.ShapeDtypeStruct((N, 8), f32),
            jax.ShapeDtypeStruct((N, 8), f32),
        ),
    )(kcat, kc, oc, tg, s0, dgo, dgi, t0h, t0f, uch, ucf, s0t, s1t)


def _cell_body(px_ref, ph_ref, h_ref, ni_ref, no_ref, wi_ref, bi_ref,
               wh_ref, bh_ref, h_out_ref, hs_out_ref):
    dot = lambda a, b: jnp.dot(a, b, preferred_element_type=jnp.float32)
    ni = ni_ref[:, 0:1]
    aggx = (px_ref[0] + px_ref[1]) * ni
    aggh = (ph_ref[0] + ph_ref[1]) * ni
    i = dot(aggx, wi_ref[...]) + bi_ref[...]
    hh = dot(aggh, wh_ref[...]) + bh_ref[...]
    r = jax.nn.sigmoid(i[:, 0:H] + hh[:, 0:H])
    z = jax.nn.sigmoid(i[:, H:2 * H] + hh[:, H:2 * H])
    n = jnp.tanh(i[:, 2 * H:] + r * hh[:, 2 * H:])
    hnew = (1.0 - z) * n + z * h_ref[...]
    h_out_ref[...] = hnew
    hs_out_ref[...] = hnew * no_ref[:, 0:1]


def _cell(px, ph, h, ni, no, wi, bi, wh, bh):
    f32 = jnp.float32
    bs = pl.BlockSpec
    return pl.pallas_call(
        _cell_body,
        grid=(NB,),
        in_specs=[
            bs((2, BN, H), lambda i: (0, i, 0)),
            bs((2, BN, H), lambda i: (0, i, 0)),
            bs((BN, H), lambda i: (i, 0)),
            bs((BN, 8), lambda i: (i, 0)),
            bs((BN, 8), lambda i: (i, 0)),
            bs((H, 3 * H), lambda i: (0, 0)),
            bs((1, 3 * H), lambda i: (0, 0)),
            bs((H, 3 * H), lambda i: (0, 0)),
            bs((1, 3 * H), lambda i: (0, 0)),
        ],
        out_specs=[
            bs((BN, H), lambda i: (i, 0)),
            bs((BN, H), lambda i: (i, 0)),
        ],
        out_shape=(
            jax.ShapeDtypeStruct((N, H), f32),
            jax.ShapeDtypeStruct((N, H), f32),
        ),
    )(px, ph, h, ni, no, wi, bi, wh, bh)


def _out_body(fh_ref, w_ref, b_ref, o_ref):
    dot = lambda a, b: jnp.dot(a, b, preferred_element_type=jnp.float32)
    for t in range(T - ENC):
        o_ref[t] = dot(fh_ref[t], w_ref[...]) + b_ref[...]


def _outproj(fh, w8, b8):
    bs = pl.BlockSpec
    return pl.pallas_call(
        _out_body,
        grid=(NB,),
        in_specs=[
            bs((T - ENC, BN, H), lambda i: (0, i, 0)),
            bs((H, 8), lambda i: (0, 0)),
            bs((1, 8), lambda i: (0, 0)),
        ],
        out_specs=bs((T - ENC, BN, 8), lambda i: (0, i, 0)),
        out_shape=jax.ShapeDtypeStruct((T - ENC, N, 8), jnp.float32),
    )(fh, w8, b8)


def _slabs(idx, padval):
    a = idx.reshape(NW, EPW)
    return jnp.pad(a, ((0, 0), (0, CPAD)), constant_values=padval)


def _pad_rows(a, rows):
    return jnp.pad(a, ((0, rows - a.shape[0]), (0, 0)))


def kernel(s_cat, k_cat, k_cont, o_cont, target, edge_index, params):
    f32 = jnp.float32
    i32 = jnp.int32
    src = edge_index[0].astype(i32)
    dst = edge_index[1].astype(i32)
    order = jnp.argsort(src)
    src = src[order]
    dst = dst[order]

    pedges = _slabs(src * PACK + dst, TRASH)
    pdego = _slabs(src * PACK + src, TRASH)
    pdegi = _slabs(dst * PACK + dst, TRASH)

    z128 = jnp.zeros((ZR, H), f32)
    ones_n = jnp.ones((N, H), f32)

    prop = _make_prop(H)

    dgo = prop(ones_n, pdego, z128)
    dgi = prop(ones_n, pdegi, z128)

    p = params
    t0h, t0f, uch, ucf, s0t, s1t = _prep(
        _pad_rows(p["k_cat_emb"][0], 56),
        p["hist_down_W"],
        p["hist_down_b"].reshape(1, H),
        p["k_cont_vec"], p["k_cont_bias"],
        p["o_cont_vec"], p["o_cont_bias"],
        p["tgt_vec"], p["tgt_bias"],
        p["fut_down_W"],
        p["fut_down_b"].reshape(1, H),
        _pad_rows(p["s_cat_emb"][0], 104),
        _pad_rows(p["s_cat_emb"][1], 104),
        p["static_W"],
        p["static_b"].reshape(1, 2 * H),
    )

    hx, fx, h0, h0s, no8, ni8 = _features(
        k_cat[:, :, 0].astype(i32),
        k_cont.reshape(N, 2 * T),
        o_cont.reshape(N, 2 * T),
        target.reshape(N, T),
        s_cat[:, 0, :].astype(i32),
        dgo, dgi, t0h, t0f, uch, ucf, s0t, s1t)

    def run_gru(layers, xs_stack, nsteps, h_list, hs_list):
        h_fin, hs_fin = [], []
        outs_h = []
        for l, lp in enumerate(layers):
            px_all = [prop(xs_stack[t], pedges, z128) for t in range(nsteps)]
            h, hs = h_list[l], hs_list[l]
            bi = lp["bi"].reshape(1, 3 * H)
            bh = lp["bh"].reshape(1, 3 * H)
            outs_h, outs_hs = [], []
            for t in range(nsteps):
                ph = prop(hs, pedges, z128)
                h, hs = _cell(px_all[t], ph, h, ni8, no8,
                              lp["Wi"], bi, lp["Wh"], bh)
                outs_h.append(h)
                outs_hs.append(hs)
            xs_stack = jnp.stack(outs_hs, axis=0)
            h_fin.append(h)
            hs_fin.append(hs)
        return jnp.stack(outs_h, axis=0), h_fin, hs_fin

    _, h_fin, hs_fin = run_gru(p["hist_layers"], hx, ENC,
                               [h0[0], h0[1]], [h0s[0], h0s[1]])
    fut_stack, _, _ = run_gru(p["fut_layers"], fx, T - ENC, h_fin, hs_fin)

    w8 = jnp.pad(p["out_W"], ((0, 0), (0, 7)))
    b8 = jnp.pad(p["out_b"], (0, 7)).reshape(1, 8)
    res = _outproj(fut_stack, w8, b8)
    return jnp.transpose(res[:, :, 0:1], (1, 0, 2))


# speedup vs baseline: 2.5152x; 2.5152x over previous
import functools

import jax
import jax.numpy as jnp
from jax import lax
from jax.experimental import pallas as pl
from jax.experimental.pallas import tpu as pltpu
from jax.experimental.pallas import tpu_sc as plsc

N = 10000
E = 320000
T = 16
ENC = 12
H = 128
NUM_LAYERS = 2

NW = 32
EPW = E // NW
CHUNK = 112
SLOTS = 3
C = -(-EPW // (SLOTS * CHUNK)) * SLOTS
PVH = C // 2
PACK = 16384
CPAD = C * CHUNK - EPW
ACC = 10112
ZR = ACC // 16
TRASH = N
BN = 1000
NB = N // BN


@functools.lru_cache(maxsize=None)
def _make_prop(W):
    mesh = plsc.VectorSubcoreMesh(core_axis_name="c", subcore_axis_name="s")

    @functools.partial(
        pl.kernel,
        mesh=mesh,
        out_type=jax.ShapeDtypeStruct((2, ACC, W), jnp.float32),
        scratch_types=[
            pltpu.VMEM((PVH * CHUNK,), jnp.int32),
            pltpu.VMEM((SLOTS * CHUNK,), jnp.int32),
            pltpu.VMEM((SLOTS, CHUNK), jnp.int32),
            pltpu.VMEM((SLOTS, CHUNK, W), jnp.float32),
            pltpu.VMEM_SHARED((ACC, W), jnp.float32),
            pltpu.SemaphoreType.DMA,
            pltpu.SemaphoreType.DMA,
            pltpu.SemaphoreType.DMA,
        ],
    )
    def prop(xs_hbm, pidx_hbm, zeros_hbm, out_hbm,
             pv, gb, sb, rows, acc, sem0, sem1, sem2):
        cid = lax.axis_index("c")
        sid = lax.axis_index("s")
        wid = sid * 2 + cid
        sems = (sem0, sem1, sem2)
        pltpu.sync_copy(pidx_hbm.at[2 * wid], pv)
        pltpu.sync_copy(zeros_hbm, acc.at[pl.ds(sid * ZR, ZR)])
        plsc.subcore_barrier()

        def unpack(j, slot):
            jh = lax.rem(j, PVH)
            for k in range(CHUNK // 16):
                v = pv[pl.ds(jh * CHUNK + 16 * k, 16)]
                gb[pl.ds(slot * CHUNK + 16 * k, 16)] = (
                    lax.shift_right_logical(v, 14))
                sb[slot, pl.ds(16 * k, 16)] = lax.bitwise_and(v, PACK - 1)

        def fire(j, slot, sem):
            if not isinstance(j, int) or j == PVH:
                @pl.when(j == PVH)
                def _():
                    pltpu.sync_copy(pidx_hbm.at[2 * wid + 1], pv)
            unpack(j, slot)
            pltpu.async_copy(
                xs_hbm.at[gb.at[pl.ds(slot * CHUNK, CHUNK)]],
                rows.at[slot], sem)

        for s in range(SLOTS):
            fire(s, s, sems[s])

        def body(i, carry):
            j0 = SLOTS * i
            for slot in range(SLOTS):
                j = j0 + slot
                sem = sems[slot]
                pltpu.make_async_copy(
                    xs_hbm.at[pl.ds(0, CHUNK)], rows.at[slot], sem).wait()
                pltpu.sync_copy(rows.at[slot], acc.at[sb.at[slot]], add=True)

                @pl.when(j + SLOTS < C)
                def _():
                    fire(j + SLOTS, slot, sem)
            return carry

        lax.fori_loop(0, C // SLOTS, body, 0)
        plsc.subcore_barrier()
        pltpu.sync_copy(acc.at[pl.ds(sid * ZR, ZR)],
                        out_hbm.at[cid, pl.ds(sid * ZR, ZR)])

    return prop


def _prep_body(kemb_ref, hdw_ref, hdb_ref, kcv_ref, kcb_ref, ocv_ref, ocb_ref,
               tgv_ref, tgb_ref, fdw_ref, fdb_ref, se0_ref, se1_ref, stw_ref,
               stb_ref, t0h_ref, t0f_ref, uch_ref, ucf_ref, s0_ref, s1_ref):
    hdw = hdw_ref[...]
    fdw = fdw_ref[...]
    dot = lambda a, b: jnp.dot(a, b, preferred_element_type=jnp.float32)
    t0h_ref[...] = dot(kemb_ref[...], hdw[0:128])
    t0f_ref[...] = dot(kemb_ref[...], fdw[0:128])
    kcv = kcv_ref[...]
    kcb = kcb_ref[...]
    ocv = ocv_ref[...]
    ocb = ocb_ref[...]
    u1h = dot(kcv[0:1], hdw[128:256])
    u2h = dot(kcv[1:2], hdw[256:384])
    u3h = dot(ocv[0:1], hdw[384:512])
    u4h = dot(ocv[1:2], hdw[512:640])
    u5h = dot(tgv_ref[...], hdw[640:768])
    ch = (dot(kcb[0:1], hdw[128:256]) + dot(kcb[1:2], hdw[256:384])
          + dot(ocb[0:1], hdw[384:512]) + dot(ocb[1:2], hdw[512:640])
          + dot(tgb_ref[...], hdw[640:768]) + hdb_ref[...])
    zrow = jnp.zeros((2, 128), jnp.float32)
    uch_ref[...] = jnp.concatenate([u1h, u2h, u3h, u4h, u5h, ch, zrow], axis=0)
    u1f = dot(kcv[0:1], fdw[128:256])
    u2f = dot(kcv[1:2], fdw[256:384])
    cf = (dot(kcb[0:1], fdw[128:256]) + dot(kcb[1:2], fdw[256:384])
          + fdb_ref[...])
    zrow5 = jnp.zeros((5, 128), jnp.float32)
    ucf_ref[...] = jnp.concatenate([u1f, u2f, cf, zrow5], axis=0)
    stw = stw_ref[...]
    s0_ref[...] = dot(se0_ref[...], stw[0:128])
    s1_ref[...] = dot(se1_ref[...], stw[128:256]) + stb_ref[...]


def _prep(kemb, hdw, hdb, kcv, kcb, ocv, ocb, tgv, tgb, fdw, fdb,
          se0, se1, stw, stb):
    f32 = jnp.float32
    return pl.pallas_call(
        _prep_body,
        out_shape=(
            jax.ShapeDtypeStruct((56, 128), f32),
            jax.ShapeDtypeStruct((56, 128), f32),
            jax.ShapeDtypeStruct((8, 128), f32),
            jax.ShapeDtypeStruct((8, 128), f32),
            jax.ShapeDtypeStruct((104, 256), f32),
            jax.ShapeDtypeStruct((104, 256), f32),
        ),
    )(kemb, hdw, hdb, kcv, kcb, ocv, ocb, tgv, tgb, fdw, fdb, se0, se1,
      stw, stb)


def _feat_body(kcat_ref, kc_ref, oc_ref, tg_ref, s0_ref, dgo_ref, dgi_ref,
               t0h_ref, t0f_ref, uch_ref, ucf_ref, s0t_ref, s1t_ref,
               hx_ref, fx_ref, h0_ref, h0s_ref, no_ref, ni_ref):
    dot = lambda a, b: jnp.dot(a, b, preferred_element_type=jnp.float32)
    no = lax.rsqrt(jnp.maximum(dgo_ref[0, :, 0] + dgo_ref[1, :, 0], 1.0))
    ni = lax.rsqrt(jnp.maximum(dgi_ref[0, :, 0] + dgi_ref[1, :, 0], 1.0))
    no = no[:, None]
    ni = ni[:, None]
    no_ref[...] = jnp.broadcast_to(no, (BN, 8))
    ni_ref[...] = jnp.broadcast_to(ni, (BN, 8))

    ids = kcat_ref[...]
    kc = kc_ref[...]
    oc = oc_ref[...]
    tg = tg_ref[...]
    uch = uch_ref[...]
    ucf = ucf_ref[...]
    iota56 = lax.broadcasted_iota(jnp.int32, (1, 56), 1)
    for t in range(T):
        oh = (ids[:, t][:, None] == iota56).astype(jnp.float32)
        if t < ENC:
            v = (dot(oh, t0h_ref[...])
                 + kc[:, 2 * t][:, None] * uch[0:1]
                 + kc[:, 2 * t + 1][:, None] * uch[1:2]
                 + oc[:, 2 * t][:, None] * uch[2:3]
                 + oc[:, 2 * t + 1][:, None] * uch[3:4]
                 + tg[:, t][:, None] * uch[4:5]
                 + uch[5:6])
            hx_ref[t] = v * no
        else:
            v = (dot(oh, t0f_ref[...])
                 + kc[:, 2 * t][:, None] * ucf[0:1]
                 + kc[:, 2 * t + 1][:, None] * ucf[1:2]
                 + ucf[2:3])
            fx_ref[t - ENC] = v * no

    s0 = s0_ref[...]
    iota104 = lax.broadcasted_iota(jnp.int32, (1, 104), 1)
    oh0 = (s0[:, 0][:, None] == iota104).astype(jnp.float32)
    oh1 = (s0[:, 1][:, None] == iota104).astype(jnp.float32)
    iv = dot(oh0, s0t_ref[...]) + dot(oh1, s1t_ref[...])
    h00 = iv[:, 0:128]
    h01 = iv[:, 128:256]
    h0_ref[0] = h00
    h0_ref[1] = h01
    h0s_ref[0] = h00 * no
    h0s_ref[1] = h01 * no


def _features(kcat, kc, oc, tg, s0, dgo, dgi, t0h, t0f, uch, ucf, s0t, s1t):
    f32 = jnp.float32
    bs = pl.BlockSpec
    return pl.pallas_call(
        _feat_body,
        grid=(NB,),
        in_specs=[
            bs((BN, T), lambda i: (i, 0)),
            bs((BN, 2 * T), lambda i: (i, 0)),
            bs((BN, 2 * T), lambda i: (i, 0)),
            bs((BN, T), lambda i: (i, 0)),
            bs((BN, 2), lambda i: (i, 0)),
            bs((2, BN, H), lambda i: (0, i, 0)),
            bs((2, BN, H), lambda i: (0, i, 0)),
            bs((56, 128), lambda i: (0, 0)),
            bs((56, 128), lambda i: (0, 0)),
            bs((8, 128), lambda i: (0, 0)),
            bs((8, 128), lambda i: (0, 0)),
            bs((104, 256), lambda i: (0, 0)),
            bs((104, 256), lambda i: (0, 0)),
        ],
        out_specs=[
            bs((ENC, BN, H), lambda i: (0, i, 0)),
            bs((T - ENC, BN, H), lambda i: (0, i, 0)),
            bs((2, BN, H), lambda i: (0, i, 0)),
            bs((2, BN, H), lambda i: (0, i, 0)),
            bs((BN, 8), lambda i: (i, 0)),
            bs((BN, 8), lambda i: (i, 0)),
        ],
        out_shape=(
            jax.ShapeDtypeStruct((ENC, N, H), f32),
            jax.ShapeDtypeStruct((T - ENC, N, H), f32),
            jax.ShapeDtypeStruct((2, N, H), f32),
            jax.ShapeDtypeStruct((2, N, H), f32),
            jax.ShapeDtypeStruct((N, 8), f32),
            jax.ShapeDtypeStruct((N, 8), f32),
        ),
    )(kcat, kc, oc, tg, s0, dgo, dgi, t0h, t0f, uch, ucf, s0t, s1t)


def _cell_body(px_ref, ph_ref, h_ref, ni_ref, no_ref, wi_ref, bi_ref,
               wh_ref, bh_ref, h_out_ref, hs_out_ref):
    dot = lambda a, b: jnp.dot(a, b, preferred_element_type=jnp.float32)
    ni = ni_ref[:, 0:1]
    aggx = (px_ref[0] + px_ref[1]) * ni
    aggh = (ph_ref[0] + ph_ref[1]) * ni
    i = dot(aggx, wi_ref[...]) + bi_ref[...]
    hh = dot(aggh, wh_ref[...]) + bh_ref[...]
    r = jax.nn.sigmoid(i[:, 0:H] + hh[:, 0:H])
    z = jax.nn.sigmoid(i[:, H:2 * H] + hh[:, H:2 * H])
    n = jnp.tanh(i[:, 2 * H:] + r * hh[:, 2 * H:])
    hnew = (1.0 - z) * n + z * h_ref[...]
    h_out_ref[...] = hnew
    hs_out_ref[...] = hnew * no_ref[:, 0:1]


def _cell(px, ph, h, ni, no, wi, bi, wh, bh):
    f32 = jnp.float32
    bs = pl.BlockSpec
    return pl.pallas_call(
        _cell_body,
        grid=(NB,),
        in_specs=[
            bs((2, BN, H), lambda i: (0, i, 0)),
            bs((2, BN, H), lambda i: (0, i, 0)),
            bs((BN, H), lambda i: (i, 0)),
            bs((BN, 8), lambda i: (i, 0)),
            bs((BN, 8), lambda i: (i, 0)),
            bs((H, 3 * H), lambda i: (0, 0)),
            bs((1, 3 * H), lambda i: (0, 0)),
            bs((H, 3 * H), lambda i: (0, 0)),
            bs((1, 3 * H), lambda i: (0, 0)),
        ],
        out_specs=[
            bs((BN, H), lambda i: (i, 0)),
            bs((BN, H), lambda i: (i, 0)),
        ],
        out_shape=(
            jax.ShapeDtypeStruct((N, H), f32),
            jax.ShapeDtypeStruct((N, H), f32),
        ),
    )(px, ph, h, ni, no, wi, bi, wh, bh)


def _out_body(fh_ref, w_ref, b_ref, o_ref):
    dot = lambda a, b: jnp.dot(a, b, preferred_element_type=jnp.float32)
    for t in range(T - ENC):
        o_ref[t] = dot(fh_ref[t], w_ref[...]) + b_ref[...]


def _outproj(fh, w8, b8):
    bs = pl.BlockSpec
    return pl.pallas_call(
        _out_body,
        grid=(NB,),
        in_specs=[
            bs((T - ENC, BN, H), lambda i: (0, i, 0)),
            bs((H, 8), lambda i: (0, 0)),
            bs((1, 8), lambda i: (0, 0)),
        ],
        out_specs=bs((T - ENC, BN, 8), lambda i: (0, i, 0)),
        out_shape=jax.ShapeDtypeStruct((T - ENC, N, 8), jnp.float32),
    )(fh, w8, b8)


def _slabs(idx, padval):
    a = idx.reshape(NW, EPW)
    a = jnp.pad(a, ((0, 0), (0, CPAD)), constant_values=padval)
    return a.reshape(NW * 2, PVH * CHUNK)


def _pad_rows(a, rows):
    return jnp.pad(a, ((0, rows - a.shape[0]), (0, 0)))


def kernel(s_cat, k_cat, k_cont, o_cont, target, edge_index, params):
    f32 = jnp.float32
    i32 = jnp.int32
    src = edge_index[0].astype(i32)
    dst = edge_index[1].astype(i32)
    pedges = _slabs(src * PACK + dst, TRASH)
    pdego = _slabs(src * PACK + src, TRASH)
    pdegi = _slabs(dst * PACK + dst, TRASH)

    z128 = jnp.zeros((ZR, H), f32)
    ones_n = jnp.ones((N, H), f32)

    prop = _make_prop(H)

    dgo = prop(ones_n, pdego, z128)
    dgi = prop(ones_n, pdegi, z128)

    p = params
    t0h, t0f, uch, ucf, s0t, s1t = _prep(
        _pad_rows(p["k_cat_emb"][0], 56),
        p["hist_down_W"],
        p["hist_down_b"].reshape(1, H),
        p["k_cont_vec"], p["k_cont_bias"],
        p["o_cont_vec"], p["o_cont_bias"],
        p["tgt_vec"], p["tgt_bias"],
        p["fut_down_W"],
        p["fut_down_b"].reshape(1, H),
        _pad_rows(p["s_cat_emb"][0], 104),
        _pad_rows(p["s_cat_emb"][1], 104),
        p["static_W"],
        p["static_b"].reshape(1, 2 * H),
    )

    hx, fx, h0, h0s, no8, ni8 = _features(
        k_cat[:, :, 0].astype(i32),
        k_cont.reshape(N, 2 * T),
        o_cont.reshape(N, 2 * T),
        target.reshape(N, T),
        s_cat[:, 0, :].astype(i32),
        dgo, dgi, t0h, t0f, uch, ucf, s0t, s1t)

    def run_gru(layers, xs_stack, nsteps, h_list, hs_list):
        h_fin, hs_fin = [], []
        outs_h = []
        for l, lp in enumerate(layers):
            px_all = [prop(xs_stack[t], pedges, z128) for t in range(nsteps)]
            h, hs = h_list[l], hs_list[l]
            bi = lp["bi"].reshape(1, 3 * H)
            bh = lp["bh"].reshape(1, 3 * H)
            outs_h, outs_hs = [], []
            for t in range(nsteps):
                ph = prop(hs, pedges, z128)
                h, hs = _cell(px_all[t], ph, h, ni8, no8,
                              lp["Wi"], bi, lp["Wh"], bh)
                outs_h.append(h)
                outs_hs.append(hs)
            xs_stack = jnp.stack(outs_hs, axis=0)
            h_fin.append(h)
            hs_fin.append(hs)
        return jnp.stack(outs_h, axis=0), h_fin, hs_fin

    _, h_fin, hs_fin = run_gru(p["hist_layers"], hx, ENC,
                               [h0[0], h0[1]], [h0s[0], h0s[1]])
    fut_stack, _, _ = run_gru(p["fut_layers"], fx, T - ENC, h_fin, hs_fin)

    w8 = jnp.pad(p["out_W"], ((0, 0), (0, 7)))
    b8 = jnp.pad(p["out_b"], (0, 7)).reshape(1, 8)
    res = _outproj(fut_stack, w8, b8)
    return jnp.transpose(res[:, :, 0:1], (1, 0, 2))
